# Initial kernel scaffold; baseline (speedup 1.0000x reference)
import jax
import jax.numpy as jnp
from jax.experimental import pallas as pl


def kernel(token, feature):
    raise NotImplementedError("write your pallas kernel here")


# speedup vs baseline: 138.5802x; 138.5802x over previous
import jax
import jax.numpy as jnp
from jax.experimental import pallas as pl

_R = 8


def _body(tok_ref, feat_ref, out_ref, *, hw, c, topk):
    tok = tok_ref[...]
    feat = feat_ref[...]
    scale = c ** -0.5
    feat2d = feat.reshape(hw * feat.shape[0], c)
    s_full = jax.lax.dot_general(
        feat2d, tok, (((1,), (1,)), ((), ())),
        precision=jax.lax.Precision.DEFAULT,
        preferred_element_type=jnp.float32)
    s3 = s_full.reshape(feat.shape[0], hw, feat.shape[0])
    rr = jax.lax.broadcasted_iota(jnp.int32, s3.shape, 0)
    ll = jax.lax.broadcasted_iota(jnp.int32, s3.shape, 2)
    s = jnp.sum(jnp.where(rr == ll, s3, 0.0), axis=-1) * scale
    m = jnp.max(s, axis=-1, keepdims=True)

    bits = jax.lax.bitcast_convert_type(s, jnp.int32)
    key = jnp.where(bits >= 0, bits, bits ^ jnp.int32(0x7FFFFFFF))

    lo = jnp.min(key, axis=-1, keepdims=True)
    hi = jnp.max(key, axis=-1, keepdims=True) + 1

    def step(_, lohi):
        lo, hi = lohi
        mid = (lo >> 1) + (hi >> 1) + (lo & hi & 1)
        cnt = jnp.sum((key >= mid).astype(jnp.int32), axis=-1,
                      keepdims=True)
        ge = cnt >= topk
        return jnp.where(ge, mid, lo), jnp.where(ge, hi, mid)

    lo, hi = jax.lax.fori_loop(0, 32, step, (lo, hi))
    w = jnp.where(key >= lo, jnp.exp(s - m), 0.0)
    denom = jnp.sum(w, axis=-1, keepdims=True)
    out = jnp.sum(feat * w[:, :, None], axis=1)
    out_ref[...] = out / denom


def kernel(token, feature):
    b, n, k, c = token.shape
    hw = feature.shape[3]
    nrows = b * n * k
    topk = int(hw * 0.5)
    tok = token.reshape(nrows, c)
    feat = feature.reshape(nrows, hw, c)

    import functools
    body = functools.partial(_body, hw=hw, c=c, topk=topk)
    out = pl.pallas_call(
        body,
        grid=(nrows // _R,),
        in_specs=[
            pl.BlockSpec((_R, c), lambda i: (i, 0)),
            pl.BlockSpec((_R, hw, c), lambda i: (i, 0, 0)),
        ],
        out_specs=pl.BlockSpec((_R, c), lambda i: (i, 0)),
        out_shape=jax.ShapeDtypeStruct((nrows, c), jnp.float32),
    )(tok, feat)
    return out.reshape(b, n, k, c)
